# Initial kernel scaffold; baseline (speedup 1.0000x reference)
import jax
import jax.numpy as jnp
from jax.experimental import pallas as pl


def kernel(x):
    raise NotImplementedError("write your pallas kernel here")


# speedup vs baseline: 9.6404x; 9.6404x over previous
import numpy as np
import jax
import jax.numpy as jnp
from jax.experimental import pallas as pl
from jax.experimental.pallas import tpu as pltpu

_BOX_SIZES = (3, 5, 7, 9, 11)
_H = 224
_PQ_THRESHOLD = 0.59275


def _box_consts(b):
    rows = -(-_H // b)
    hp = rows * b
    top = (hp - _H) // 2
    c = (b - 1) // 2
    g = np.zeros((rows, _H), np.float32)
    g[np.arange(rows), np.arange(rows) * b + c - top] = 1.0
    s = np.zeros((rows, _H), np.float32)
    s[(np.arange(_H) + top) // b, np.arange(_H)] = 1.0
    nreal = s.sum(axis=1)
    npad = (np.float32(b * b) - np.outer(nreal, nreal)).astype(np.float32)
    return g, s, npad


_CONSTS = tuple(_box_consts(b) for b in _BOX_SIZES)


def _fnn_kernel(x_ref, *refs):
    out_ref = refs[-1]
    mats = refs[:-1]
    f32 = jnp.float32
    dn = jax.lax.dot_general
    lane = jax.lax.broadcasted_iota(jnp.int32, (1, 45), 1)
    row = jnp.zeros((1, 45), f32)

    x0 = x_ref[0, 0]
    x1 = x_ref[0, 1]
    x2 = x_ref[0, 2]
    norm2 = x0 * x0 + x1 * x1 + x2 * x2

    for bi, b in enumerate(_BOX_SIZES):
        g = mats[3 * bi][...]
        s = mats[3 * bi + 1][...]
        npad = mats[3 * bi + 2][...]
        lcap = f32(b * b)
        thr = f32(b)
        rows = g.shape[0]
        nump = f32(rows * rows)

        cen_grid = []
        cen_pix = []
        for xc in (x0, x1, x2):
            t = dn(g, xc, (((1,), (0,)), ((), ())),
                   preferred_element_type=f32)
            u = dn(t, g, (((1,), (1,)), ((), ())),
                   preferred_element_type=f32)
            v = dn(s, u, (((0,), (0,)), ((), ())),
                   preferred_element_type=f32)
            cp = dn(v, s, (((1,), (0,)), ((), ())),
                    preferred_element_type=f32)
            cen_grid.append(u)
            cen_pix.append(cp)

        d0 = jnp.abs(x0 - cen_pix[0])
        d1 = jnp.abs(x1 - cen_pix[1])
        d2 = jnp.abs(x2 - cen_pix[2])
        cheb = (jnp.maximum(jnp.maximum(d0, d1), d2) <= thr).astype(f32)
        manh = ((d0 + d1 + d2) <= thr).astype(f32)
        eucl = (norm2 <= thr * thr).astype(f32)

        a0 = jnp.abs(cen_grid[0])
        a1 = jnp.abs(cen_grid[1])
        a2 = jnp.abs(cen_grid[2])
        padb_cheb = (jnp.maximum(jnp.maximum(a0, a1), a2) <= thr).astype(f32)
        padb_manh = ((a0 + a1 + a2) <= thr).astype(f32)
        padb_eucl = jnp.ones_like(npad)

        for d, (binimg, padb) in enumerate(
                ((cheb, padb_cheb), (eucl, padb_eucl), (manh, padb_manh))):
            a = dn(s, binimg, (((1,), (0,)), ((), ())),
                   preferred_element_type=f32)
            core = dn(a, s, (((1,), (1,)), ((), ())),
                      preferred_element_type=f32)
            counts = core + npad * padb
            cp1 = counts + 1.0
            valid = (counts < lcap - 0.5).astype(f32)
            s0 = jnp.sum(valid)
            sinv = jnp.sum(valid / cp1)
            s1 = jnp.sum(valid * cp1)
            s2 = jnp.sum(valid * cp1 * cp1)
            pq = jnp.sum((counts / lcap >= _PQ_THRESHOLD).astype(f32)) / nump
            fd = sinv / s0
            mu1 = s1 / s0
            mu1sq = mu1 * mu1
            lac = (s2 / s0 - mu1sq) / mu1sq
            idx = d * 15 + bi * 3
            row = row + jnp.where(lane == idx, fd, 0.0)
            row = row + jnp.where(lane == idx + 1, lac, 0.0)
            row = row + jnp.where(lane == idx + 2, pq, 0.0)

    out_ref[0] = row


def kernel(x):
    bsz = x.shape[0]
    xt = x.transpose(0, 3, 1, 2)

    args = [xt]
    in_specs = [pl.BlockSpec((1, 3, _H, _H), lambda i: (i, 0, 0, 0))]
    for g, s, npad in _CONSTS:
        for m in (g, s, npad):
            args.append(jnp.asarray(m))
            in_specs.append(pl.BlockSpec(m.shape, lambda i: (0, 0)))

    out = pl.pallas_call(
        _fnn_kernel,
        grid=(bsz,),
        in_specs=in_specs,
        out_specs=pl.BlockSpec((1, 1, 45), lambda i: (i, 0, 0)),
        out_shape=jax.ShapeDtypeStruct((bsz, 1, 45), jnp.float32),
        compiler_params=pltpu.CompilerParams(
            dimension_semantics=("parallel",)),
    )(*args)
    return out.reshape(bsz, 45)


# speedup vs baseline: 21.1116x; 2.1899x over previous
import numpy as np
import jax
import jax.numpy as jnp
from jax.experimental import pallas as pl
from jax.experimental.pallas import tpu as pltpu

_BOX_SIZES = (3, 5, 7, 9, 11)
_H = 224
_PQ_THRESHOLD = 0.59275


def _box_consts(b):
    rows = -(-_H // b)
    hp = rows * b
    top = (hp - _H) // 2
    c = (b - 1) // 2
    g = np.zeros((rows, _H), np.float32)
    g[np.arange(rows), np.arange(rows) * b + c - top] = 1.0
    s = np.zeros((rows, _H), np.float32)
    s[(np.arange(_H) + top) // b, np.arange(_H)] = 1.0
    nreal = s.sum(axis=1)
    npad = (np.float32(b * b) - np.outer(nreal, nreal)).astype(np.float32)
    return g, s, npad


_CONSTS = tuple(_box_consts(b) for b in _BOX_SIZES)
_NB = len(_BOX_SIZES)


def _image_row(x_ref, j, mats):
    f32 = jnp.float32
    dn = jax.lax.dot_general
    lane = jax.lax.broadcasted_iota(jnp.int32, (1, 45), 1)
    row = jnp.zeros((1, 45), f32)

    x0 = x_ref[j, 0]
    x1 = x_ref[j, 1]
    x2 = x_ref[j, 2]
    xs = (x0, x1, x2)
    norm2 = x0 * x0 + x1 * x1 + x2 * x2

    gs = [mats[3 * bi][...] for bi in range(_NB)]
    ss = [mats[3 * bi + 1][...] for bi in range(_NB)]
    npads = [mats[3 * bi + 2][...] for bi in range(_NB)]

    crs = []
    for bi, b in enumerate(_BOX_SIZES):
        rows_b = gs[bi].shape[0]
        hp = rows_b * b
        top = (hp - _H) // 2
        bot = hp - _H - top
        cc = (b - 1) // 2
        per_ch = []
        for xc in xs:
            if top or bot:
                xcp = jnp.concatenate(
                    ([jnp.zeros((top, _H), f32)] if top else [])
                    + [xc]
                    + ([jnp.zeros((bot, _H), f32)] if bot else []), axis=0)
            else:
                xcp = xc
            per_ch.append(xcp.reshape(rows_b, b, _H)[:, cc, :])
        crs.append(per_ch)

    us = [[dn(crs[bi][c], gs[bi], (((1,), (1,)), ((), ())),
              preferred_element_type=f32)
           for c in range(3)] for bi in range(_NB)]

    vs = [[dn(us[bi][c], ss[bi], (((1,), (0,)), ((), ())),
              preferred_element_type=f32)
           for c in range(3)] for bi in range(_NB)]

    cfs = [[dn(ss[bi], vs[bi][c], (((0,), (0,)), ((), ())),
               preferred_element_type=f32)
            for c in range(3)] for bi in range(_NB)]

    rss = []
    for bi, b in enumerate(_BOX_SIZES):
        thr = f32(b)
        d0 = jnp.abs(x0 - cfs[bi][0])
        d1 = jnp.abs(x1 - cfs[bi][1])
        d2 = jnp.abs(x2 - cfs[bi][2])
        cheb = (jnp.maximum(jnp.maximum(d0, d1), d2) <= thr).astype(f32)
        manh = ((d0 + d1 + d2) <= thr).astype(f32)
        eucl = (norm2 <= thr * thr).astype(f32)
        rss.append([dn(ss[bi], binimg, (((1,), (0,)), ((), ())),
                       preferred_element_type=f32)
                    for binimg in (cheb, eucl, manh)])

    cores = [[dn(rss[bi][d], ss[bi], (((1,), (1,)), ((), ())),
                 preferred_element_type=f32)
              for d in range(3)] for bi in range(_NB)]

    for bi, b in enumerate(_BOX_SIZES):
        lcap = f32(b * b)
        thr = f32(b)
        rows_b = gs[bi].shape[0]
        nump = f32(rows_b * rows_b)
        npad = npads[bi]
        a0 = jnp.abs(us[bi][0])
        a1 = jnp.abs(us[bi][1])
        a2 = jnp.abs(us[bi][2])
        padb_cheb = (jnp.maximum(jnp.maximum(a0, a1), a2) <= thr).astype(f32)
        padb_manh = ((a0 + a1 + a2) <= thr).astype(f32)
        padb_eucl = jnp.ones_like(npad)
        for d, padb in enumerate((padb_cheb, padb_eucl, padb_manh)):
            counts = cores[bi][d] + npad * padb
            cp1 = counts + 1.0
            valid = (counts < lcap - 0.5).astype(f32)
            s0 = jnp.sum(valid)
            sinv = jnp.sum(valid / cp1)
            s1 = jnp.sum(valid * cp1)
            s2 = jnp.sum(valid * cp1 * cp1)
            pq = jnp.sum((counts / lcap >= _PQ_THRESHOLD).astype(f32)) / nump
            fd = sinv / s0
            mu1 = s1 / s0
            mu1sq = mu1 * mu1
            lac = (s2 / s0 - mu1sq) / mu1sq
            idx = d * 15 + bi * 3
            row = row + jnp.where(lane == idx, fd, 0.0)
            row = row + jnp.where(lane == idx + 1, lac, 0.0)
            row = row + jnp.where(lane == idx + 2, pq, 0.0)

    return row


def _fnn_kernel(x_ref, *refs):
    out_ref = refs[-1]
    mats = refs[:-1]
    rows = [_image_row(x_ref, j, mats) for j in range(x_ref.shape[0])]
    out_ref[0] = jnp.concatenate(rows, axis=0) if len(rows) > 1 else rows[0]


_IMGS_PER_PROG = 1


def kernel(x):
    bsz = x.shape[0]
    xt = x.transpose(0, 3, 1, 2)
    ipp = _IMGS_PER_PROG
    ngrid = bsz // ipp

    args = [xt]
    in_specs = [pl.BlockSpec((ipp, 3, _H, _H), lambda i: (i, 0, 0, 0))]
    for g, s, npad in _CONSTS:
        for m in (g, s, npad):
            args.append(jnp.asarray(m))
            in_specs.append(pl.BlockSpec(m.shape, lambda i: (0, 0)))

    out = pl.pallas_call(
        _fnn_kernel,
        grid=(ngrid,),
        in_specs=in_specs,
        out_specs=pl.BlockSpec((1, ipp, 45), lambda i: (i, 0, 0)),
        out_shape=jax.ShapeDtypeStruct((ngrid, ipp, 45), jnp.float32),
        compiler_params=pltpu.CompilerParams(
            dimension_semantics=("parallel",)),
    )(*args)
    return out.reshape(bsz, 45)


# speedup vs baseline: 25.4129x; 1.2037x over previous
import numpy as np
import jax
import jax.numpy as jnp
from jax.experimental import pallas as pl
from jax.experimental.pallas import tpu as pltpu

_BOX_SIZES = (3, 5, 7, 9, 11)
_H = 224
_PQ_THRESHOLD = 0.59275


def _box_consts(b):
    rows = -(-_H // b)
    hp = rows * b
    top = (hp - _H) // 2
    c = (b - 1) // 2
    g = np.zeros((rows, _H), np.float32)
    g[np.arange(rows), np.arange(rows) * b + c - top] = 1.0
    s = np.zeros((rows, _H), np.float32)
    s[(np.arange(_H) + top) // b, np.arange(_H)] = 1.0
    nreal = s.sum(axis=1)
    npad = (np.float32(b * b) - np.outer(nreal, nreal)).astype(np.float32)
    return g, s, npad


_CONSTS = tuple(_box_consts(b) for b in _BOX_SIZES)
_NB = len(_BOX_SIZES)


def _image_row(x_ref, j, mats):
    f32 = jnp.float32
    dn = jax.lax.dot_general
    lane = jax.lax.broadcasted_iota(jnp.int32, (1, 45), 1)
    row = jnp.zeros((1, 45), f32)

    x0 = x_ref[j, 0]
    x1 = x_ref[j, 1]
    x2 = x_ref[j, 2]
    xs = (x0, x1, x2)
    norm2 = x0 * x0 + x1 * x1 + x2 * x2

    gs = [mats[3 * bi][...] for bi in range(_NB)]
    ss = [mats[3 * bi + 1][...] for bi in range(_NB)]
    npads = [mats[3 * bi + 2][...] for bi in range(_NB)]

    crs = []
    for bi, b in enumerate(_BOX_SIZES):
        rows_b = gs[bi].shape[0]
        hp = rows_b * b
        top = (hp - _H) // 2
        bot = hp - _H - top
        cc = (b - 1) // 2
        per_ch = []
        for xc in xs:
            if top or bot:
                xcp = jnp.concatenate(
                    ([jnp.zeros((top, _H), f32)] if top else [])
                    + [xc]
                    + ([jnp.zeros((bot, _H), f32)] if bot else []), axis=0)
            else:
                xcp = xc
            per_ch.append(xcp.reshape(rows_b, b, _H)[:, cc, :])
        crs.append(per_ch)

    us = [[dn(crs[bi][c], gs[bi], (((1,), (1,)), ((), ())),
              preferred_element_type=f32)
           for c in range(3)] for bi in range(_NB)]

    vs = [[dn(us[bi][c], ss[bi], (((1,), (0,)), ((), ())),
              preferred_element_type=f32)
           for c in range(3)] for bi in range(_NB)]

    cfs = [[dn(ss[bi], vs[bi][c], (((0,), (0,)), ((), ())),
               preferred_element_type=f32)
            for c in range(3)] for bi in range(_NB)]

    rss = []
    for bi, b in enumerate(_BOX_SIZES):
        thr = f32(b)
        d0 = jnp.abs(x0 - cfs[bi][0])
        d1 = jnp.abs(x1 - cfs[bi][1])
        d2 = jnp.abs(x2 - cfs[bi][2])
        cheb = (jnp.maximum(jnp.maximum(d0, d1), d2) <= thr).astype(f32)
        manh = ((d0 + d1 + d2) <= thr).astype(f32)
        eucl = (norm2 <= thr * thr).astype(f32)
        rss.append([dn(ss[bi], binimg, (((1,), (0,)), ((), ())),
                       preferred_element_type=f32)
                    for binimg in (cheb, eucl, manh)])

    cores = [[dn(rss[bi][d], ss[bi], (((1,), (1,)), ((), ())),
                 preferred_element_type=f32)
              for d in range(3)] for bi in range(_NB)]

    for bi, b in enumerate(_BOX_SIZES):
        lcap = f32(b * b)
        thr = f32(b)
        rows_b = gs[bi].shape[0]
        nump = f32(rows_b * rows_b)
        npad = npads[bi]
        a0 = jnp.abs(us[bi][0])
        a1 = jnp.abs(us[bi][1])
        a2 = jnp.abs(us[bi][2])
        padb_cheb = (jnp.maximum(jnp.maximum(a0, a1), a2) <= thr).astype(f32)
        padb_manh = ((a0 + a1 + a2) <= thr).astype(f32)
        padb_eucl = jnp.ones_like(npad)
        for d, padb in enumerate((padb_cheb, padb_eucl, padb_manh)):
            counts = cores[bi][d] + npad * padb
            cp1 = counts + 1.0
            valid = (counts < lcap - 0.5).astype(f32)
            s0 = jnp.sum(valid)
            sinv = jnp.sum(valid / cp1)
            s1 = jnp.sum(valid * cp1)
            s2 = jnp.sum(valid * cp1 * cp1)
            pq = jnp.sum((counts / lcap >= _PQ_THRESHOLD).astype(f32)) / nump
            fd = sinv / s0
            mu1 = s1 / s0
            mu1sq = mu1 * mu1
            lac = (s2 / s0 - mu1sq) / mu1sq
            idx = d * 15 + bi * 3
            row = row + jnp.where(lane == idx, fd, 0.0)
            row = row + jnp.where(lane == idx + 1, lac, 0.0)
            row = row + jnp.where(lane == idx + 2, pq, 0.0)

    return row


def _fnn_kernel(x_ref, *refs):
    out_ref = refs[-1]
    mats = refs[:-1]
    rows = [_image_row(x_ref, j, mats) for j in range(x_ref.shape[0])]
    out_ref[0] = jnp.concatenate(rows, axis=0) if len(rows) > 1 else rows[0]


_IMGS_PER_PROG = 2


def kernel(x):
    bsz = x.shape[0]
    xt = x.transpose(0, 3, 1, 2)
    ipp = _IMGS_PER_PROG
    ngrid = bsz // ipp

    args = [xt]
    in_specs = [pl.BlockSpec((ipp, 3, _H, _H), lambda i: (i, 0, 0, 0))]
    for g, s, npad in _CONSTS:
        for m in (g, s, npad):
            args.append(jnp.asarray(m))
            in_specs.append(pl.BlockSpec(m.shape, lambda i: (0, 0)))

    out = pl.pallas_call(
        _fnn_kernel,
        grid=(ngrid,),
        in_specs=in_specs,
        out_specs=pl.BlockSpec((1, ipp, 45), lambda i: (i, 0, 0)),
        out_shape=jax.ShapeDtypeStruct((ngrid, ipp, 45), jnp.float32),
        compiler_params=pltpu.CompilerParams(
            dimension_semantics=("parallel",)),
    )(*args)
    return out.reshape(bsz, 45)


# speedup vs baseline: 30.4572x; 1.1985x over previous
import numpy as np
import jax
import jax.numpy as jnp
from jax.experimental import pallas as pl
from jax.experimental.pallas import tpu as pltpu

_BOX_SIZES = (3, 5, 7, 9, 11)
_H = 224
_PQ_THRESHOLD = 0.59275


def _box_consts(b):
    rows = -(-_H // b)
    hp = rows * b
    top = (hp - _H) // 2
    c = (b - 1) // 2
    g = np.zeros((rows, _H), np.float32)
    g[np.arange(rows), np.arange(rows) * b + c - top] = 1.0
    s = np.zeros((rows, _H), np.float32)
    s[(np.arange(_H) + top) // b, np.arange(_H)] = 1.0
    nreal = s.sum(axis=1)
    npad = (np.float32(b * b) - np.outer(nreal, nreal)).astype(np.float32)
    return g, s, npad


_CONSTS = tuple(_box_consts(b) for b in _BOX_SIZES)
_NB = len(_BOX_SIZES)


def _image_row(x_ref, j, mats):
    f32 = jnp.float32
    dn = jax.lax.dot_general
    lane = jax.lax.broadcasted_iota(jnp.int32, (1, 45), 1)
    row = jnp.zeros((1, 45), f32)

    x0 = x_ref[j, 0]
    x1 = x_ref[j, 1]
    x2 = x_ref[j, 2]
    xs = (x0, x1, x2)
    norm2 = x0 * x0 + x1 * x1 + x2 * x2

    gs = [mats[3 * bi][...] for bi in range(_NB)]
    ss = [mats[3 * bi + 1][...] for bi in range(_NB)]
    npads = [mats[3 * bi + 2][...] for bi in range(_NB)]

    def cen_rows(bi):
        b = _BOX_SIZES[bi]
        rows_b = gs[bi].shape[0]
        hp = rows_b * b
        top = (hp - _H) // 2
        bot = hp - _H - top
        cc = (b - 1) // 2
        crs = []
        for xc in xs:
            if top or bot:
                xcp = jnp.concatenate(
                    ([jnp.zeros((top, _H), f32)] if top else [])
                    + [xc]
                    + ([jnp.zeros((bot, _H), f32)] if bot else []), axis=0)
            else:
                xcp = xc
            crs.append(xcp.reshape(rows_b, b, _H)[:, cc, :])
        return crs

    def binaries(bi, cf):
        thr = f32(_BOX_SIZES[bi])
        d0 = jnp.abs(x0 - cf[0])
        d1 = jnp.abs(x1 - cf[1])
        d2 = jnp.abs(x2 - cf[2])
        cheb = (jnp.maximum(jnp.maximum(d0, d1), d2) <= thr).astype(f32)
        manh = ((d0 + d1 + d2) <= thr).astype(f32)
        eucl = (norm2 <= thr * thr).astype(f32)
        return cheb, eucl, manh

    def stats(bi, u, cores, row):
        b = _BOX_SIZES[bi]
        lcap = f32(b * b)
        thr = f32(b)
        rows_b = gs[bi].shape[0]
        nump = f32(rows_b * rows_b)
        npad = npads[bi]
        a0 = jnp.abs(u[0])
        a1 = jnp.abs(u[1])
        a2 = jnp.abs(u[2])
        padb_cheb = (jnp.maximum(jnp.maximum(a0, a1), a2) <= thr).astype(f32)
        padb_manh = ((a0 + a1 + a2) <= thr).astype(f32)
        padb_eucl = jnp.ones_like(npad)
        for d, padb in enumerate((padb_cheb, padb_eucl, padb_manh)):
            counts = cores[d] + npad * padb
            cp1 = counts + 1.0
            valid = (counts < lcap - 0.5).astype(f32)
            s0 = jnp.sum(valid)
            sinv = jnp.sum(valid / cp1)
            s1 = jnp.sum(valid * cp1)
            s2 = jnp.sum(valid * cp1 * cp1)
            pq = jnp.sum((counts / lcap >= _PQ_THRESHOLD).astype(f32)) / nump
            fd = sinv / s0
            mu1 = s1 / s0
            mu1sq = mu1 * mu1
            lac = (s2 / s0 - mu1sq) / mu1sq
            idx = d * 15 + bi * 3
            row = row + jnp.where(lane == idx, fd, 0.0)
            row = row + jnp.where(lane == idx + 1, lac, 0.0)
            row = row + jnp.where(lane == idx + 2, pq, 0.0)
        return row

    crss = [cen_rows(bi) for bi in range(_NB)]
    us = [[dn(cr, gs[bi], (((1,), (1,)), ((), ())),
              preferred_element_type=f32) for cr in crss[bi]]
          for bi in range(_NB)]
    vss = [[dn(uc, ss[bi], (((1,), (0,)), ((), ())),
               preferred_element_type=f32) for uc in us[bi]]
           for bi in range(_NB)]
    cfs = [[dn(ss[bi], vc, (((0,), (0,)), ((), ())),
               preferred_element_type=f32) for vc in vss[bi]]
           for bi in range(_NB)]
    rss = []
    for bi in range(_NB):
        bins = binaries(bi, cfs[bi])
        rss.append([dn(ss[bi], binimg, (((1,), (0,)), ((), ())),
                       preferred_element_type=f32) for binimg in bins])
    cores = [[dn(r, ss[bi], (((1,), (1,)), ((), ())),
                 preferred_element_type=f32) for r in rss[bi]]
             for bi in range(_NB)]
    for bi in range(_NB):
        row = stats(bi, us[bi], cores[bi], row)

    return row


def _fnn_kernel(x_ref, *refs):
    out_ref = refs[-1]
    mats = refs[:-1]
    rows = [_image_row(x_ref, j, mats) for j in range(x_ref.shape[0])]
    out_ref[0] = jnp.concatenate(rows, axis=0) if len(rows) > 1 else rows[0]


_IMGS_PER_PROG = 8


def kernel(x):
    bsz = x.shape[0]
    xt = x.transpose(0, 3, 1, 2)
    ipp = _IMGS_PER_PROG
    ngrid = bsz // ipp

    args = [xt]
    in_specs = [pl.BlockSpec((ipp, 3, _H, _H), lambda i: (i, 0, 0, 0))]
    for g, s, npad in _CONSTS:
        for m in (g, s, npad):
            args.append(jnp.asarray(m))
            in_specs.append(pl.BlockSpec(m.shape, lambda i: (0, 0)))

    out = pl.pallas_call(
        _fnn_kernel,
        grid=(ngrid,),
        in_specs=in_specs,
        out_specs=pl.BlockSpec((1, ipp, 45), lambda i: (i, 0, 0)),
        out_shape=jax.ShapeDtypeStruct((ngrid, ipp, 45), jnp.float32),
        compiler_params=pltpu.CompilerParams(
            dimension_semantics=("parallel",)),
    )(*args)
    return out.reshape(bsz, 45)


# speedup vs baseline: 30.8491x; 1.0129x over previous
import numpy as np
import jax
import jax.numpy as jnp
from jax.experimental import pallas as pl
from jax.experimental.pallas import tpu as pltpu

_BOX_SIZES = (3, 5, 7, 9, 11)
_H = 224
_PQ_THRESHOLD = 0.59275


def _box_consts(b):
    rows = -(-_H // b)
    hp = rows * b
    top = (hp - _H) // 2
    c = (b - 1) // 2
    g = np.zeros((rows, _H), np.float32)
    g[np.arange(rows), np.arange(rows) * b + c - top] = 1.0
    s = np.zeros((rows, _H), np.float32)
    s[(np.arange(_H) + top) // b, np.arange(_H)] = 1.0
    nreal = s.sum(axis=1)
    npad = (np.float32(b * b) - np.outer(nreal, nreal)).astype(np.float32)
    return g, s, npad


_CONSTS = tuple(_box_consts(b) for b in _BOX_SIZES)
_NB = len(_BOX_SIZES)


def _image_row(x_ref, j, mats):
    f32 = jnp.float32
    dn = jax.lax.dot_general
    lane = jax.lax.broadcasted_iota(jnp.int32, (1, 45), 1)
    row = jnp.zeros((1, 45), f32)

    x0 = x_ref[j, 0]
    x1 = x_ref[j, 1]
    x2 = x_ref[j, 2]
    xs = (x0, x1, x2)
    norm2 = x0 * x0 + x1 * x1 + x2 * x2

    gs = [mats[3 * bi][...] for bi in range(_NB)]
    ss = [mats[3 * bi + 1][...] for bi in range(_NB)]
    npads = [mats[3 * bi + 2][...] for bi in range(_NB)]

    def cen_rows(bi):
        return [dn(gs[bi], xc, (((1,), (0,)), ((), ())),
                   preferred_element_type=f32) for xc in xs]

    def binaries(bi, cf):
        thr = f32(_BOX_SIZES[bi])
        d0 = jnp.abs(x0 - cf[0])
        d1 = jnp.abs(x1 - cf[1])
        d2 = jnp.abs(x2 - cf[2])
        cheb = (jnp.maximum(jnp.maximum(d0, d1), d2) <= thr).astype(f32)
        manh = ((d0 + d1 + d2) <= thr).astype(f32)
        eucl = (norm2 <= thr * thr).astype(f32)
        return cheb, eucl, manh

    def stats(bi, u, cores, row):
        b = _BOX_SIZES[bi]
        lcap = f32(b * b)
        thr = f32(b)
        rows_b = gs[bi].shape[0]
        nump = f32(rows_b * rows_b)
        npad = npads[bi]
        a0 = jnp.abs(u[0])
        a1 = jnp.abs(u[1])
        a2 = jnp.abs(u[2])
        padb_cheb = (jnp.maximum(jnp.maximum(a0, a1), a2) <= thr).astype(f32)
        padb_manh = ((a0 + a1 + a2) <= thr).astype(f32)
        padb_eucl = jnp.ones_like(npad)
        for d, padb in enumerate((padb_cheb, padb_eucl, padb_manh)):
            counts = cores[d] + npad * padb
            cp1 = counts + 1.0
            valid = (counts < lcap - 0.5).astype(f32)
            s0 = jnp.sum(valid)
            sinv = jnp.sum(valid / cp1)
            s1 = jnp.sum(valid * cp1)
            s2 = jnp.sum(valid * cp1 * cp1)
            pq = jnp.sum((counts / lcap >= _PQ_THRESHOLD).astype(f32)) / nump
            fd = sinv / s0
            mu1 = s1 / s0
            mu1sq = mu1 * mu1
            lac = (s2 / s0 - mu1sq) / mu1sq
            idx = d * 15 + bi * 3
            row = row + jnp.where(lane == idx, fd, 0.0)
            row = row + jnp.where(lane == idx + 1, lac, 0.0)
            row = row + jnp.where(lane == idx + 2, pq, 0.0)
        return row

    crss = [cen_rows(bi) for bi in range(_NB)]
    us = [[dn(cr, gs[bi], (((1,), (1,)), ((), ())),
              preferred_element_type=f32) for cr in crss[bi]]
          for bi in range(_NB)]
    vss = [[dn(uc, ss[bi], (((1,), (0,)), ((), ())),
               preferred_element_type=f32) for uc in us[bi]]
           for bi in range(_NB)]
    cfs = [[dn(ss[bi], vc, (((0,), (0,)), ((), ())),
               preferred_element_type=f32) for vc in vss[bi]]
           for bi in range(_NB)]
    rss = []
    for bi in range(_NB):
        bins = binaries(bi, cfs[bi])
        rss.append([dn(ss[bi], binimg, (((1,), (0,)), ((), ())),
                       preferred_element_type=f32) for binimg in bins])
    cores = [[dn(r, ss[bi], (((1,), (1,)), ((), ())),
                 preferred_element_type=f32) for r in rss[bi]]
             for bi in range(_NB)]
    for bi in range(_NB):
        row = stats(bi, us[bi], cores[bi], row)

    return row


def _fnn_kernel(x_ref, *refs):
    out_ref = refs[-1]
    mats = refs[:-1]
    rows = [_image_row(x_ref, j, mats) for j in range(x_ref.shape[0])]
    out_ref[0] = jnp.concatenate(rows, axis=0) if len(rows) > 1 else rows[0]


_IMGS_PER_PROG = 8


def kernel(x):
    bsz = x.shape[0]
    xt = x.transpose(0, 3, 1, 2)
    ipp = _IMGS_PER_PROG
    ngrid = bsz // ipp

    args = [xt]
    in_specs = [pl.BlockSpec((ipp, 3, _H, _H), lambda i: (i, 0, 0, 0))]
    for g, s, npad in _CONSTS:
        for m in (g, s, npad):
            args.append(jnp.asarray(m))
            in_specs.append(pl.BlockSpec(m.shape, lambda i: (0, 0)))

    out = pl.pallas_call(
        _fnn_kernel,
        grid=(ngrid,),
        in_specs=in_specs,
        out_specs=pl.BlockSpec((1, ipp, 45), lambda i: (i, 0, 0)),
        out_shape=jax.ShapeDtypeStruct((ngrid, ipp, 45), jnp.float32),
        compiler_params=pltpu.CompilerParams(
            dimension_semantics=("parallel",)),
    )(*args)
    return out.reshape(bsz, 45)
